# Initial kernel scaffold; baseline (speedup 1.0000x reference)
import jax
import jax.numpy as jnp
from jax.experimental import pallas as pl


def kernel(inputs, probs, training):
    raise NotImplementedError("write your pallas kernel here")


# speedup vs baseline: 12.7042x; 12.7042x over previous
import functools

import jax
import jax.numpy as jnp
from jax import lax
from jax.experimental import pallas as pl
from jax.experimental.pallas import tpu as pltpu

_B = 64
_N = 32768
_RB = 8


def _mask_kernel(kidx_ref, x_ref, o_ref, keys_ref):
    x = x_ref[...]
    kb = lax.bitcast_convert_type(x, jnp.int32)
    keys = jnp.where(kb < 0, kb ^ jnp.int32(0x7FFFFFFF), kb)
    keys_ref[...] = keys
    k = kidx_ref[:, 0:1]

    sign = jnp.int32(-2147483648)

    def body(i, p_u):
        bit = lax.shift_left(jnp.int32(1), jnp.int32(31) - i)
        cand_u = p_u | bit
        cand_s = cand_u ^ sign
        cnt = jnp.sum((keys_ref[...] < cand_s).astype(jnp.int32), axis=1,
                      keepdims=True)
        return jnp.where(cnt <= k, cand_u, p_u)

    p_u = lax.fori_loop(0, 32, body, jnp.zeros((_RB, 1), jnp.int32))
    thr_s = p_u ^ sign
    o_ref[...] = jnp.where(keys_ref[...] < thr_s, jnp.float32(0.0), x)


def kernel(inputs, probs, training):
    n = inputs.shape[-1]
    kidx = jnp.maximum(
        jnp.ceil(jnp.float32(n) * probs).astype(jnp.int32) - 1, 0)
    kidx = jnp.where(training != 0, kidx, 0)
    kidx2 = jnp.broadcast_to(kidx[:, None], (_B, 128))

    out = pl.pallas_call(
        _mask_kernel,
        grid=(_B // _RB,),
        in_specs=[
            pl.BlockSpec((_RB, 128), lambda i: (i, 0)),
            pl.BlockSpec((_RB, _N), lambda i: (i, 0)),
        ],
        out_specs=pl.BlockSpec((_RB, _N), lambda i: (i, 0)),
        out_shape=jax.ShapeDtypeStruct((_B, _N), jnp.float32),
        scratch_shapes=[pltpu.VMEM((_RB, _N), jnp.int32)],
    )(kidx2, inputs)
    return out
